# baseline (speedup 1.0000x reference)
import functools

import jax
import jax.numpy as jnp
from jax.experimental import pallas as pl
from jax.experimental.pallas import tpu as pltpu

G_HEAD = 12
V_CLUSTER = 1024
HEAD_SIZE = 32
AUG = HEAD_SIZE + 2


def _pq_kernel(x_ref, e_ref, q_ref, c_ref, e2_ref, eaug_ref):
    tn = x_ref.shape[0]

    @pl.when(pl.program_id(0) == 0)
    def _fill():
        e = e_ref[...]
        e2_ref[...] = jnp.sum(e * e, axis=2)
        idx = jax.lax.broadcasted_iota(
            jnp.int32, (G_HEAD, V_CLUSTER, 1), 1).astype(jnp.float32)
        ones = jnp.ones((G_HEAD, V_CLUSTER, 1), jnp.float32)
        eaug_ref[...] = jnp.concatenate([e, idx, ones], axis=2)

    xgs = [x_ref[:, g * HEAD_SIZE:(g + 1) * HEAD_SIZE] for g in range(G_HEAD)]
    us = []
    for g in range(G_HEAD):
        us.append(jax.lax.dot_general(
            xgs[g] + xgs[g], eaug_ref[g, :, :HEAD_SIZE],
            (((1,), (1,)), ((), ())),
            preferred_element_type=jnp.float32))
    masks = []
    for g in range(G_HEAD):
        w = us[g] - e2_ref[g:g + 1, :]
        m = jnp.max(w, axis=1, keepdims=True)
        masks.append((w == m).astype(jnp.float32))
    outs = []
    for g in range(G_HEAD):
        outs.append(jax.lax.dot_general(
            masks[g], eaug_ref[g], (((1,), (0,)), ((), ())),
            preferred_element_type=jnp.float32))
    q_parts = []
    c_parts = []
    for g in range(G_HEAD):
        out = outs[g]
        inv = 1.0 / out[:, HEAD_SIZE + 1:HEAD_SIZE + 2]
        qg = out[:, :HEAD_SIZE] * inv
        codes = (out[:, HEAD_SIZE:HEAD_SIZE + 1] * inv).astype(jnp.int32)
        q_parts.append(xgs[g] + (qg - xgs[g]))
        c_parts.append(codes)
    q_ref[...] = jnp.concatenate(q_parts, axis=1)
    c_ref[...] = jnp.concatenate(c_parts, axis=1)


@functools.partial(jax.jit, static_argnames=("tile",))
def kernel(input, embed, *, tile=512):
    B, T, n_embed = input.shape
    gH, K, Hs = embed.shape
    BT = B * T
    x2d = input.reshape(BT, n_embed)
    grid = (BT // tile,)
    q2d, c2d = pl.pallas_call(
        _pq_kernel,
        grid=grid,
        in_specs=[
            pl.BlockSpec((tile, n_embed), lambda i: (i, 0)),
            pl.BlockSpec((gH, K, Hs), lambda i: (0, 0, 0)),
        ],
        out_specs=[
            pl.BlockSpec((tile, n_embed), lambda i: (i, 0)),
            pl.BlockSpec((tile, gH), lambda i: (i, 0)),
        ],
        out_shape=[
            jax.ShapeDtypeStruct((BT, n_embed), jnp.float32),
            jax.ShapeDtypeStruct((BT, gH), jnp.int32),
        ],
        scratch_shapes=[
            pltpu.VMEM((gH, K), jnp.float32),
            pltpu.VMEM((gH, K, AUG), jnp.float32),
        ],
        compiler_params=pltpu.CompilerParams(
            dimension_semantics=("arbitrary",)),
    )(x2d, embed)
    return q2d.reshape(B, T, n_embed), c2d.reshape(B, T, gH)
